# Initial kernel scaffold; baseline (speedup 1.0000x reference)
import jax
import jax.numpy as jnp
from jax.experimental import pallas as pl


def kernel(X, edge_index, Wl0, bl0, Wr0, Wl1, bl1, Wr1, W0, b0, W1, b1, W2, b2):
    raise NotImplementedError("write your pallas kernel here")


# speedup vs baseline: 14.3174x; 14.3174x over previous
import functools

import jax
import jax.numpy as jnp
from jax import lax
from jax.experimental import pallas as pl
from jax.experimental.pallas import tpu as pltpu
from jax.experimental.pallas import tpu_sc as plsc

N = 1000
NPAD = 1024
E = 64000
NC = 2
NS = 16
L = 16
NW = NC * NS
EPW = E // NW
GROUPS = EPW // L
NEG = -3.0e38

_f32 = jnp.float32


_GATHER_DN = lax.GatherDimensionNumbers(
    offset_dims=(), collapsed_slice_dims=(0,), start_index_map=(0,))


def _take(x, idx):
    return lax.gather(x, idx[:, None], _GATHER_DN, (1,),
                      mode=lax.GatherScatterMode.PROMISE_IN_BOUNDS)


_mesh = plsc.VectorSubcoreMesh(core_axis_name="c", subcore_axis_name="s")


@functools.partial(
    pl.kernel,
    out_type=[jax.ShapeDtypeStruct((NW, NPAD), _f32)] * 5,
    mesh=_mesh,
    compiler_params=pltpu.CompilerParams(needs_layout_passes=False),
    scratch_types=[
        pltpu.VMEM((EPW,), jnp.int32),
        pltpu.VMEM((EPW,), jnp.int32),
        pltpu.VMEM((NPAD,), _f32),
        pltpu.VMEM((NPAD,), _f32),
        pltpu.VMEM((NPAD,), _f32),
        pltpu.VMEM((NPAD,), _f32),
        pltpu.VMEM((NPAD,), _f32),
        pltpu.VMEM((NPAD,), _f32),
        pltpu.VMEM((NPAD,), _f32),
    ],
)
def _sc_aggregate(src_hbm, dst_hbm, x0_hbm, x1_hbm,
                  o_sum0, o_sum1, o_cnt, o_max0, o_max1,
                  src_v, dst_v, x0_v, x1_v,
                  sum0_v, sum1_v, cnt_v, max0_v, max1_v):
    wid = lax.axis_index("s") * NC + lax.axis_index("c")
    base = wid * EPW
    pltpu.sync_copy(src_hbm.at[pl.ds(base, EPW)], src_v)
    pltpu.sync_copy(dst_hbm.at[pl.ds(base, EPW)], dst_v)
    pltpu.sync_copy(x0_hbm, x0_v)
    pltpu.sync_copy(x1_hbm, x1_v)

    zeros16 = jnp.zeros((L,), _f32)
    neg16 = jnp.full((L,), NEG, _f32)

    def init_body(j, c):
        off = j * L
        sum0_v[pl.ds(off, L)] = zeros16
        sum1_v[pl.ds(off, L)] = zeros16
        cnt_v[pl.ds(off, L)] = zeros16
        max0_v[pl.ds(off, L)] = neg16
        max1_v[pl.ds(off, L)] = neg16
        return c
    lax.fori_loop(0, NPAD // L, init_body, 0)

    lanes = lax.iota(jnp.int32, L)

    def group_body(g, c):
        off = g * L
        d = dst_v[pl.ds(off, L)]
        s = src_v[pl.ds(off, L)]
        ds_, ss = plsc.sort_key_val(d, s)
        g0 = plsc.load_gather(x0_v, [ss])
        g1 = plsc.load_gather(x1_v, [ss])
        seg0, seg1 = g0, g1
        segc = jnp.ones((L,), _f32)
        m0, m1 = g0, g1
        for step in (1, 2, 4, 8):
            pidx = jnp.maximum(lanes - step, 0)
            same = (_take(ds_, pidx) == ds_) & (lanes >= step)
            seg0 = seg0 + jnp.where(same, _take(seg0, pidx), 0.0)
            seg1 = seg1 + jnp.where(same, _take(seg1, pidx), 0.0)
            segc = segc + jnp.where(same, _take(segc, pidx), 0.0)
            m0 = jnp.where(same, jnp.maximum(m0, _take(m0, pidx)), m0)
            m1 = jnp.where(same, jnp.maximum(m1, _take(m1, pidx)), m1)
        is_last = (_take(ds_, jnp.minimum(lanes + 1, L - 1)) != ds_) | (lanes == L - 1)
        plsc.addupdate_scatter(sum0_v, [ds_], seg0, mask=is_last)
        plsc.addupdate_scatter(sum1_v, [ds_], seg1, mask=is_last)
        plsc.addupdate_scatter(cnt_v, [ds_], segc, mask=is_last)
        c0 = plsc.load_gather(max0_v, [ds_])
        c1 = plsc.load_gather(max1_v, [ds_])
        plsc.store_scatter(max0_v, [ds_], jnp.maximum(c0, m0), mask=is_last)
        plsc.store_scatter(max1_v, [ds_], jnp.maximum(c1, m1), mask=is_last)
        return c
    lax.fori_loop(0, GROUPS, group_body, 0)

    pltpu.sync_copy(sum0_v, o_sum0.at[wid])
    pltpu.sync_copy(sum1_v, o_sum1.at[wid])
    pltpu.sync_copy(cnt_v, o_cnt.at[wid])
    pltpu.sync_copy(max0_v, o_max0.at[wid])
    pltpu.sync_copy(max1_v, o_max1.at[wid])


def _tc_body(coef_ref, p_sum0, p_sum1, p_cnt, p_max0, p_max1,
             x0_ref, x1_ref, w0_ref, b0_ref, w1_ref, b1_ref, w2_ref, b2_ref,
             emb_ref, out_ref):
    dims = (((1,), (1,)), ((), ()))
    sum0 = jnp.sum(p_sum0[...], axis=0, keepdims=True)
    sum1 = jnp.sum(p_sum1[...], axis=0, keepdims=True)
    cnt = jnp.sum(p_cnt[...], axis=0, keepdims=True)
    max0 = jnp.max(p_max0[...], axis=0, keepdims=True)
    max1 = jnp.max(p_max1[...], axis=0, keepdims=True)
    denom = jnp.maximum(cnt, 1.0)
    mean0 = sum0 / denom
    mean1 = sum1 / denom
    has = cnt > 0.0
    mx0 = jnp.where(has, max0, 0.0)
    mx1 = jnp.where(has, max1, 0.0)
    x0 = x0_ref[...]
    x1 = x1_ref[...]
    c = coef_ref
    xm = jnp.maximum(mean0 * c[0] + mean1 * c[1] + c[2] + x0 * c[3] + x1 * c[4], 0.0)
    xx = jnp.maximum(mx0 * c[5] + mx1 * c[6] + c[7] + x0 * c[8] + x1 * c[9], 0.0)
    emb = xm + xx
    emb = jnp.where(lax.broadcasted_iota(jnp.int32, (1, NPAD), 1) < N, emb, 0.0)
    emb_ref[...] = emb
    e = emb[:, :N]
    h = jnp.maximum(
        lax.dot_general(e, w0_ref[...], dims, preferred_element_type=_f32)
        + b0_ref[...], 0.0)
    h = jnp.maximum(
        lax.dot_general(h, w1_ref[...], dims, preferred_element_type=_f32)
        + b1_ref[...], 0.0)
    o = jnp.maximum(
        lax.dot_general(h, w2_ref[...], dims, preferred_element_type=_f32)
        + b2_ref[...], 0.0)
    out_ref[...] = o


_mlp_call = pl.pallas_call(
    _tc_body,
    out_shape=[
        jax.ShapeDtypeStruct((1, NPAD), _f32),
        jax.ShapeDtypeStruct((1, N - 1), _f32),
    ],
    in_specs=[pl.BlockSpec(memory_space=pltpu.SMEM)] + [pl.BlockSpec()] * 13,
)


def kernel(X, edge_index, Wl0, bl0, Wr0, Wl1, bl1, Wr1, W0, b0, W1, b1, W2, b2):
    src = edge_index[0]
    dst = edge_index[1]
    x0 = jnp.zeros((NPAD,), _f32).at[:N].set(X[:, 0])
    x1 = jnp.zeros((NPAD,), _f32).at[:N].set(X[:, 1])
    coef = jnp.concatenate([
        Wl0[0], bl0, Wr0[0], Wl1[0], bl1, Wr1[0], jnp.zeros((6,), _f32)])
    s0, s1, ct, m0, m1 = _sc_aggregate(src, dst, x0, x1)
    emb, out = _mlp_call(
        coef, s0, s1, ct, m0, m1,
        x0.reshape(1, NPAD), x1.reshape(1, NPAD),
        W0, b0.reshape(1, N), W1, b1.reshape(1, N), W2, b2.reshape(1, N - 1))
    return emb[0, :N], out[0]


# speedup vs baseline: 15.4353x; 1.0781x over previous
import functools

import jax
import jax.numpy as jnp
from jax import lax
from jax.experimental import pallas as pl
from jax.experimental.pallas import tpu as pltpu
from jax.experimental.pallas import tpu_sc as plsc

N = 1000
NPAD = 1024
E = 64000
NC = 2
NS = 16
L = 16
NW = NC * NS
EPW = E // NW
GROUPS = EPW // L
NEG = -3.0e38

_f32 = jnp.float32

_GATHER_DN = lax.GatherDimensionNumbers(
    offset_dims=(), collapsed_slice_dims=(0,), start_index_map=(0,))


def _take(x, idx):
    return lax.gather(x, idx[:, None], _GATHER_DN, (1,),
                      mode=lax.GatherScatterMode.PROMISE_IN_BOUNDS)


_mesh = plsc.VectorSubcoreMesh(core_axis_name="c", subcore_axis_name="s")


@functools.partial(
    pl.kernel,
    out_type=[jax.ShapeDtypeStruct((NW, NPAD), _f32)] * 5,
    mesh=_mesh,
    compiler_params=pltpu.CompilerParams(needs_layout_passes=False),
    scratch_types=[
        pltpu.VMEM((EPW,), jnp.int32),
        pltpu.VMEM((EPW,), jnp.int32),
        pltpu.VMEM((2 * N,), _f32),
        pltpu.VMEM((NPAD,), _f32),
        pltpu.VMEM((NPAD,), _f32),
        pltpu.VMEM((NPAD,), _f32),
        pltpu.VMEM((NPAD,), _f32),
        pltpu.VMEM((NPAD,), _f32),
    ],
)
def _sc_aggregate(edge_hbm, x_hbm,
                  o_sum0, o_sum1, o_cnt, o_max0, o_max1,
                  src_v, dst_v, x_v,
                  sum0_v, sum1_v, cnt_v, max0_v, max1_v):
    wid = lax.axis_index("s") * NC + lax.axis_index("c")
    base = wid * EPW
    pltpu.sync_copy(edge_hbm.at[pl.ds(base, EPW)], src_v)
    pltpu.sync_copy(edge_hbm.at[pl.ds(E + base, EPW)], dst_v)
    pltpu.sync_copy(x_hbm, x_v)

    zeros16 = jnp.zeros((L,), _f32)
    ones16 = jnp.ones((L,), _f32)
    neg16 = jnp.full((L,), NEG, _f32)

    def init_body(j, c):
        off = j * L
        sum0_v[pl.ds(off, L)] = zeros16
        sum1_v[pl.ds(off, L)] = zeros16
        cnt_v[pl.ds(off, L)] = zeros16
        max0_v[pl.ds(off, L)] = neg16
        max1_v[pl.ds(off, L)] = neg16
        return c
    lax.fori_loop(0, NPAD // L, init_body, 0)

    lanes = lax.iota(jnp.int32, L)

    def group_body(g, c):
        off = g * L
        d = dst_v[pl.ds(off, L)]
        s = src_v[pl.ds(off, L)]
        s2 = s + s
        g0 = plsc.load_gather(x_v, [s2])
        g1 = plsc.load_gather(x_v, [s2 + 1])
        plsc.addupdate_scatter(sum0_v, [d], g0)
        plsc.addupdate_scatter(sum1_v, [d], g1)
        plsc.addupdate_scatter(cnt_v, [d], ones16)
        ds_, m0 = plsc.sort_key_val(d, g0)
        _, m1 = plsc.sort_key_val(d, g1)
        for step in (1, 2, 4, 8):
            pidx = jnp.maximum(lanes - step, 0)
            same = (_take(ds_, pidx) == ds_) & (lanes >= step)
            m0 = jnp.where(same, jnp.maximum(m0, _take(m0, pidx)), m0)
            m1 = jnp.where(same, jnp.maximum(m1, _take(m1, pidx)), m1)
        is_last = (_take(ds_, jnp.minimum(lanes + 1, L - 1)) != ds_) | (lanes == L - 1)
        c0 = plsc.load_gather(max0_v, [ds_])
        c1 = plsc.load_gather(max1_v, [ds_])
        plsc.store_scatter(max0_v, [ds_], jnp.maximum(c0, m0), mask=is_last)
        plsc.store_scatter(max1_v, [ds_], jnp.maximum(c1, m1), mask=is_last)
        return c
    lax.fori_loop(0, GROUPS, group_body, 0)

    pltpu.sync_copy(sum0_v, o_sum0.at[wid])
    pltpu.sync_copy(sum1_v, o_sum1.at[wid])
    pltpu.sync_copy(cnt_v, o_cnt.at[wid])
    pltpu.sync_copy(max0_v, o_max0.at[wid])
    pltpu.sync_copy(max1_v, o_max1.at[wid])


def _tc_body(wl_ref, p_sum0, p_sum1, p_cnt, p_max0, p_max1,
             x_ref, wr0_ref, wr1_ref,
             w0_ref, b0_ref, w1_ref, b1_ref, w2_ref, b2_ref,
             emb_ref, out_ref):
    dims = (((1,), (1,)), ((), ()))
    sum0 = jnp.sum(p_sum0[...], axis=0, keepdims=True)[:, :N]
    sum1 = jnp.sum(p_sum1[...], axis=0, keepdims=True)[:, :N]
    cnt = jnp.sum(p_cnt[...], axis=0, keepdims=True)[:, :N]
    max0 = jnp.max(p_max0[...], axis=0, keepdims=True)[:, :N]
    max1 = jnp.max(p_max1[...], axis=0, keepdims=True)[:, :N]
    denom = jnp.maximum(cnt, 1.0)
    mean0 = sum0 / denom
    mean1 = sum1 / denom
    has = cnt > 0.0
    mx0 = jnp.where(has, max0, 0.0)
    mx1 = jnp.where(has, max1, 0.0)
    x = x_ref[...]
    xr0 = lax.dot_general(wr0_ref[...], x, dims, preferred_element_type=_f32)
    xr1 = lax.dot_general(wr1_ref[...], x, dims, preferred_element_type=_f32)
    xm = jnp.maximum(
        mean0 * wl_ref[0] + mean1 * wl_ref[1] + wl_ref[2] + xr0, 0.0)
    xx = jnp.maximum(
        mx0 * wl_ref[3] + mx1 * wl_ref[4] + wl_ref[5] + xr1, 0.0)
    emb = xm + xx
    emb_ref[...] = emb
    h = jnp.maximum(
        lax.dot_general(emb, w0_ref[...], dims, preferred_element_type=_f32)
        + b0_ref[...], 0.0)
    h = jnp.maximum(
        lax.dot_general(h, w1_ref[...], dims, preferred_element_type=_f32)
        + b1_ref[...], 0.0)
    o = jnp.maximum(
        lax.dot_general(h, w2_ref[...], dims, preferred_element_type=_f32)
        + b2_ref[...], 0.0)
    out_ref[...] = o


_mlp_call = pl.pallas_call(
    _tc_body,
    out_shape=[
        jax.ShapeDtypeStruct((1, N), _f32),
        jax.ShapeDtypeStruct((1, N - 1), _f32),
    ],
    in_specs=[pl.BlockSpec(memory_space=pltpu.SMEM)] + [pl.BlockSpec()] * 14,
)


def kernel(X, edge_index, Wl0, bl0, Wr0, Wl1, bl1, Wr1, W0, b0, W1, b1, W2, b2):
    wl = jnp.concatenate([Wl0[0], bl0, Wl1[0], bl1]).reshape(6)
    s0, s1, ct, m0, m1 = _sc_aggregate(edge_index.reshape(2 * E), X.reshape(2 * N))
    emb, out = _mlp_call(
        wl, s0, s1, ct, m0, m1,
        X, Wr0, Wr1,
        W0, b0.reshape(1, N), W1, b1.reshape(1, N), W2, b2.reshape(1, N - 1))
    return emb[0], out[0]
